# baseline (speedup 1.0000x reference)
import functools

import jax
import jax.numpy as jnp
from jax import lax
from jax.experimental import pallas as pl
from jax.experimental.pallas import tpu as pltpu
from jax.experimental.pallas import tpu_sc as plsc

_B = 4
_S = 8192
_F = 1024
_NC = 2
_NS = 16
_NW = _NC * _NS
_RPW = _S // _NW
_R = 16
_NBLK = _RPW // _R
_CH = _R * _F
_U = 8


def _sc_body(in_hbm, pe_hbm, out_hbm, pe_v, in_v):
    c = lax.axis_index("c")
    s = lax.axis_index("s")
    wid = s * _NC + c
    base = wid * _RPW * _F

    def blk(i, carry):
        off = base + i * _CH
        pltpu.sync_copy(pe_hbm.at[pl.ds(off, _CH)], pe_v)
        for b in range(_B):
            pltpu.sync_copy(
                in_hbm.at[pl.ds(b * _S * _F + off, _CH)],
                in_v.at[pl.ds(b * _CH, _CH)],
            )

        def chunk(j, carry2):
            for u in range(_U):
                o = (j * _U + u) * 16
                pv = pe_v[pl.ds(o, 16)]
                for b in range(_B):
                    in_v[pl.ds(b * _CH + o, 16)] = (
                        in_v[pl.ds(b * _CH + o, 16)] + pv
                    )
            return carry2

        lax.fori_loop(0, _CH // (16 * _U), chunk, 0)

        for b in range(_B):
            pltpu.sync_copy(
                in_v.at[pl.ds(b * _CH, _CH)],
                out_hbm.at[pl.ds(b * _S * _F + off, _CH)],
            )
        return carry

    lax.fori_loop(0, _NBLK, blk, 0)


_sc_add = functools.partial(
    pl.kernel,
    out_type=jax.ShapeDtypeStruct((_B * _S * _F,), jnp.float32),
    mesh=plsc.VectorSubcoreMesh(core_axis_name="c", subcore_axis_name="s"),
    scratch_types=[
        pltpu.VMEM((_CH,), jnp.float32),
        pltpu.VMEM((_B * _CH,), jnp.float32),
    ],
)(_sc_body)


def kernel(inputs, positional_encoding):
    b, s, f = inputs.shape
    out = _sc_add(inputs.reshape(-1), positional_encoding[:s].reshape(-1))
    return out.reshape(b, s, f)


# speedup vs baseline: 1.1582x; 1.1582x over previous
import functools

import jax
import jax.numpy as jnp
from jax import lax
from jax.experimental import pallas as pl
from jax.experimental.pallas import tpu as pltpu
from jax.experimental.pallas import tpu_sc as plsc

_B = 4
_S = 8192
_F = 1024
_NC = 2
_NS = 16
_NW = _NC * _NS
_RPW = _S // _NW
_R = 8
_NBLK = _RPW // _R
_CH = _R * _F
_U = 4
_NSET = 3


def _sc_body(in_hbm, pe_hbm, out_hbm, *scratch):
    pe_v = list(scratch[0:_NSET])
    in_v = list(scratch[_NSET:2 * _NSET])
    sin = list(scratch[2 * _NSET:3 * _NSET])
    sout = list(scratch[3 * _NSET:4 * _NSET])

    wid = lax.axis_index("s") * _NC + lax.axis_index("c")
    base = wid * _RPW * _F

    def issue_in(i, p):
        off = base + i * _CH
        hs = [pltpu.async_copy(pe_hbm.at[pl.ds(off, _CH)], pe_v[p], sin[p])]
        for b in range(_B):
            hs.append(pltpu.async_copy(
                in_hbm.at[pl.ds(b * _S * _F + off, _CH)],
                in_v[p].at[pl.ds(b * _CH, _CH)],
                sin[p],
            ))
        return hs

    def issue_out(i, p):
        off = base + i * _CH
        return [pltpu.async_copy(
            in_v[p].at[pl.ds(b * _CH, _CH)],
            out_hbm.at[pl.ds(b * _S * _F + off, _CH)],
            sout[p],
        ) for b in range(_B)]

    hin = [None] * _NSET
    hout = [None] * _NSET
    hin[0] = issue_in(0, 0)
    hin[1] = issue_in(1, 1)

    for i in range(_NBLK):
        p = i % _NSET
        for h in hin[p]:
            h.wait()
        nxt = i + 2
        if nxt < _NBLK:
            p2 = nxt % _NSET
            if hout[p2] is not None:
                for h in hout[p2]:
                    h.wait()
                hout[p2] = None
            hin[p2] = issue_in(nxt, p2)

        def chunk(j, carry, p=p):
            for u in range(_U):
                o = (j * _U + u) * 16
                pv = pe_v[p][pl.ds(o, 16)]
                for b in range(_B):
                    in_v[p][pl.ds(b * _CH + o, 16)] = (
                        in_v[p][pl.ds(b * _CH + o, 16)] + pv
                    )
            return carry

        lax.fori_loop(0, _CH // (16 * _U), chunk, 0)
        hout[p] = issue_out(i, p)

    for hs in hout:
        if hs is not None:
            for h in hs:
                h.wait()


_sc_add = functools.partial(
    pl.kernel,
    out_type=jax.ShapeDtypeStruct((_B * _S * _F,), jnp.float32),
    mesh=plsc.VectorSubcoreMesh(core_axis_name="c", subcore_axis_name="s"),
    scratch_types=(
        [pltpu.VMEM((_CH,), jnp.float32) for _ in range(_NSET)]
        + [pltpu.VMEM((_B * _CH,), jnp.float32) for _ in range(_NSET)]
        + [pltpu.SemaphoreType.DMA for _ in range(2 * _NSET)]
    ),
)(_sc_body)


def kernel(inputs, positional_encoding):
    b, s, f = inputs.shape
    out = _sc_add(inputs.reshape(-1), positional_encoding[:s].reshape(-1))
    return out.reshape(b, s, f)


# speedup vs baseline: 3.3502x; 2.8927x over previous
import functools

import jax
import jax.numpy as jnp
from jax import lax
from jax.experimental import pallas as pl
from jax.experimental.pallas import tpu as pltpu
from jax.experimental.pallas import tpu_sc as plsc

_B = 4
_S = 8192
_F = 1024
_NC = 2
_NS = 16
_NW = _NC * _NS
_RPW = _S // _NW
_R = 8
_NBLK = _RPW // _R
_NSET = 3


def _sc_body(in_hbm, pe_hbm, out_hbm, *scratch):
    pe_v = list(scratch[0:_NSET])
    in_v = list(scratch[_NSET:2 * _NSET])
    sin = list(scratch[2 * _NSET:3 * _NSET])
    sout = list(scratch[3 * _NSET:4 * _NSET])

    wid = lax.axis_index("s") * _NC + lax.axis_index("c")
    base = wid * _RPW

    def issue_in(i, p):
        r0 = base + i * _R
        hs = [pltpu.async_copy(pe_hbm.at[pl.ds(r0, _R)], pe_v[p], sin[p])]
        for b in range(_B):
            hs.append(pltpu.async_copy(
                in_hbm.at[pl.ds(b * _S + r0, _R)],
                in_v[p].at[pl.ds(b * _R, _R)],
                sin[p],
            ))
        return hs

    def issue_out(i, p):
        r0 = base + i * _R
        return [pltpu.async_copy(
            in_v[p].at[pl.ds(b * _R, _R)],
            out_hbm.at[pl.ds(b * _S + r0, _R)],
            sout[p],
        ) for b in range(_B)]

    hin = [None] * _NSET
    hout = [None] * _NSET
    hin[0] = issue_in(0, 0)
    hin[1] = issue_in(1, 1)

    for i in range(_NBLK):
        p = i % _NSET
        for h in hin[p]:
            h.wait()
        nxt = i + 2
        if nxt < _NBLK:
            p2 = nxt % _NSET
            if hout[p2] is not None:
                for h in hout[p2]:
                    h.wait()
                hout[p2] = None
            hin[p2] = issue_in(nxt, p2)

        def chunk(j, carry, p=p):
            o = j * 16
            for r in range(_R):
                pv = pe_v[p][r, pl.ds(o, 16)]
                for b in range(_B):
                    in_v[p][b * _R + r, pl.ds(o, 16)] = (
                        in_v[p][b * _R + r, pl.ds(o, 16)] + pv
                    )
            return carry

        lax.fori_loop(0, _F // 16, chunk, 0)
        hout[p] = issue_out(i, p)

    for hs in hout:
        if hs is not None:
            for h in hs:
                h.wait()


_sc_add = functools.partial(
    pl.kernel,
    out_type=jax.ShapeDtypeStruct((_B * _S, _F), jnp.float32),
    mesh=plsc.VectorSubcoreMesh(core_axis_name="c", subcore_axis_name="s"),
    compiler_params=pltpu.CompilerParams(use_tc_tiling_on_sc=True),
    scratch_types=(
        [pltpu.VMEM((_R, _F), jnp.float32) for _ in range(_NSET)]
        + [pltpu.VMEM((_B * _R, _F), jnp.float32) for _ in range(_NSET)]
        + [pltpu.SemaphoreType.DMA for _ in range(2 * _NSET)]
    ),
)(_sc_body)


def kernel(inputs, positional_encoding):
    b, s, f = inputs.shape
    out = _sc_add(inputs.reshape(b * s, f), positional_encoding[:s])
    return out.reshape(b, s, f)
